# baseline (speedup 1.0000x reference)
import functools

import jax
import jax.numpy as jnp
from jax import lax
from jax.experimental import pallas as pl
from jax.experimental.pallas import tpu as pltpu
from jax.experimental.pallas import tpu_sc as plsc

EMB = 256
N_ROWS = 16384
BETA = 0.25

ROW_BLK = 512
K_BLK = 1024

SC_WORKERS = 32
SC_CHUNK = 256


def _dist_multi_body(x_ref, *refs, kdims):
    nd = len(kdims)
    d_refs = refs[:nd]
    xn_ref = refs[nd]
    dn_refs = refs[nd + 1:2 * nd + 1]
    idx_refs = refs[2 * nd + 1:]
    xm2 = (x_ref[...] * (-2.0)).astype(jnp.bfloat16)
    xn = xn_ref[...]
    for j in range(nd):
        kdim = kdims[j]
        kb = min(K_BLK, kdim)

        def chunk_score(ci):
            sim2 = lax.dot_general(
                xm2, d_refs[j][:, ci * kb:(ci + 1) * kb],
                (((1,), (0,)), ((), ())),
                preferred_element_type=jnp.float32)
            return (xn + dn_refs[j][0:1, ci * kb:(ci + 1) * kb]) + sim2

        nchunks = kdim // kb

        def fold_pair(ci):
            if ci + 1 < nchunks:
                v1 = chunk_score(ci)
                v2 = chunk_score(ci + 1)
                c12 = jnp.where(v2 < v1, jnp.int32(ci + 1), jnp.int32(ci))
                return jnp.minimum(v1, v2), c12, 2
            return chunk_score(ci), None, 1

        m_vec, c_vec, step = fold_pair(0)
        ci = step
        while ci < nchunks:
            v, c12, step = fold_pair(ci)
            if c12 is None:
                c12 = jnp.full((ROW_BLK, kb), ci, jnp.int32)
            better = v < m_vec
            m_vec = jnp.where(better, v, m_vec)
            if c_vec is None:
                c_vec = jnp.where(better, c12, jnp.int32(0))
            else:
                c_vec = jnp.where(better, c12, c_vec)
            ci += step
        m = jnp.min(m_vec, axis=1, keepdims=True)
        kcand = lax.broadcasted_iota(jnp.int32, (ROW_BLK, kb), 1)
        if c_vec is not None:
            kcand = c_vec * jnp.int32(kb) + kcand
        a = jnp.min(jnp.where(m_vec == m, kcand, jnp.int32(2 ** 30)),
                    axis=1, keepdims=True)
        idx_refs[j][...] = jnp.reshape(a, (ROW_BLK,))


def _dist_multi(xf, ds, xn, dns):
    kdims = tuple(d.shape[1] for d in ds)
    nd = len(ds)
    return pl.pallas_call(
        functools.partial(_dist_multi_body, kdims=kdims),
        grid=(N_ROWS // ROW_BLK,),
        in_specs=(
            [pl.BlockSpec((ROW_BLK, EMB), lambda i: (i, 0))]
            + [pl.BlockSpec((EMB, k), lambda i: (0, 0)) for k in kdims]
            + [pl.BlockSpec((ROW_BLK, 1), lambda i: (i, 0))]
            + [pl.BlockSpec((1, k), lambda i: (0, 0)) for k in kdims]
        ),
        out_specs=[pl.BlockSpec((ROW_BLK,), lambda i: (i,))] * nd,
        out_shape=[jax.ShapeDtypeStruct((N_ROWS,), jnp.int32)] * nd,
    )(xf, *ds, xn, *dns)


def _sc_gather(table, idx):
    b_per_w = N_ROWS // SC_WORKERS
    mesh = plsc.VectorSubcoreMesh(core_axis_name="c", subcore_axis_name="s")

    @functools.partial(
        pl.kernel, mesh=mesh,
        out_type=jax.ShapeDtypeStruct((N_ROWS, EMB), jnp.float32),
        scratch_types=[
            pltpu.VMEM((SC_CHUNK,), jnp.int32),
            pltpu.VMEM((SC_CHUNK, EMB), jnp.float32),
            pltpu.SemaphoreType.DMA,
        ],
    )
    def k(table_hbm, idx_hbm, out_hbm, idx_v, rows_v, sem):
        wid = lax.axis_index("s") * 2 + lax.axis_index("c")
        base = wid * b_per_w
        for c in range(0, b_per_w, SC_CHUNK):
            pltpu.sync_copy(idx_hbm.at[pl.ds(base + c, SC_CHUNK)], idx_v)
            pltpu.async_copy(table_hbm.at[idx_v], rows_v, sem).wait()
            pltpu.sync_copy(rows_v, out_hbm.at[pl.ds(base + c, SC_CHUNK)])

    return k(table, idx)


def _combine_body(gam_ref, al_ref, x_ref, g0, g1, g2, g3, g4,
                  out_ref, loss_ref):
    i = pl.program_id(0)
    al = al_ref[0]
    x = x_ref[...]
    acc = jnp.zeros(x.shape, jnp.float32)
    lsum = jnp.float32(0.0)
    for j, g_ref in enumerate((g0, g1, g2, g3, g4)):
        q = al * g_ref[...].astype(jnp.bfloat16).astype(jnp.float32)
        acc = acc + gam_ref[j] * q
        dif = x - q
        lsum = lsum + gam_ref[j] * jnp.sum(dif * dif)
    out_ref[...] = acc

    @pl.when(i == 0)
    def _():
        loss_ref[...] = jnp.zeros((1, 1), jnp.float32)

    loss_ref[...] += jnp.reshape(lsum * ((1.0 + BETA) / (N_ROWS * EMB)),
                                 (1, 1))


def _combine(xf, gs, vq_gamma, vq_alpha):
    blk = 1024
    grid = (N_ROWS // blk,)
    row_spec = pl.BlockSpec((blk, EMB), lambda i: (i, 0))
    out, loss = pl.pallas_call(
        _combine_body,
        grid=grid,
        in_specs=[
            pl.BlockSpec(memory_space=pltpu.SMEM),
            pl.BlockSpec(memory_space=pltpu.SMEM),
            row_spec, row_spec, row_spec, row_spec, row_spec, row_spec,
        ],
        out_specs=[
            pl.BlockSpec((blk, EMB), lambda i: (i, 0)),
            pl.BlockSpec((1, 1), lambda i: (0, 0)),
        ],
        out_shape=[
            jax.ShapeDtypeStruct((N_ROWS, EMB), jnp.float32),
            jax.ShapeDtypeStruct((1, 1), jnp.float32),
        ],
    )(vq_gamma, vq_alpha, xf, *gs)
    return out, loss


def kernel(x, dict0, dict1, dict2, dict3, dict4, vq_alpha, vq_gamma):
    dicts = [dict0, dict1, dict2, dict3, dict4]
    xf = x.reshape(-1, EMB)
    xn = jnp.sum(xf ** 2, axis=1, keepdims=True)
    dns = [jnp.sum(d ** 2, axis=0, keepdims=True) for d in dicts]
    dbs = [d.astype(jnp.bfloat16) for d in dicts]
    idx4, idx3, idx2 = _dist_multi(
        xf, [dbs[4], dbs[3], dbs[2]], xn, [dns[4], dns[3], dns[2]])
    g4 = _sc_gather(dicts[4].T, idx4)
    g3 = _sc_gather(dicts[3].T, idx3)
    g2 = _sc_gather(dicts[2].T, idx2)
    idx1, idx0 = _dist_multi(xf, [dbs[1], dbs[0]], xn, [dns[1], dns[0]])
    g1 = _sc_gather(dicts[1].T, idx1)
    g0 = _sc_gather(dicts[0].T, idx0)
    out, loss = _combine(xf, [g0, g1, g2, g3, g4], vq_gamma,
                         vq_alpha.reshape(1))
    return out.reshape(x.shape), loss[0, 0]


# speedup vs baseline: 1.0567x; 1.0567x over previous
import functools

import jax
import jax.numpy as jnp
from jax import lax
from jax.experimental import pallas as pl
from jax.experimental.pallas import tpu as pltpu
from jax.experimental.pallas import tpu_sc as plsc

EMB = 256
N_ROWS = 16384
BETA = 0.25

ROW_BLK = 512
K_BLK = 1024

SC_WORKERS = 32
SC_CHUNK = 256


def _dist_multi_body(x_ref, *refs, kdims):
    nd = len(kdims)
    d_refs = refs[:nd]
    xn_ref = refs[nd]
    dn_refs = refs[nd + 1:2 * nd + 1]
    idx_refs = refs[2 * nd + 1:]
    xm2 = (x_ref[...] * (-2.0)).astype(jnp.bfloat16)
    xn = xn_ref[...]
    for j in range(nd):
        kdim = kdims[j]
        kb = min(K_BLK, kdim)

        def chunk_score(ci):
            sim2 = lax.dot_general(
                xm2, d_refs[j][:, ci * kb:(ci + 1) * kb],
                (((1,), (0,)), ((), ())),
                preferred_element_type=jnp.float32)
            return (xn + dn_refs[j][0:1, ci * kb:(ci + 1) * kb]) + sim2

        nchunks = kdim // kb

        def fold_pair(ci):
            if ci + 1 < nchunks:
                v1 = chunk_score(ci)
                v2 = chunk_score(ci + 1)
                c12 = jnp.where(v2 < v1, jnp.int32(ci + 1), jnp.int32(ci))
                return jnp.minimum(v1, v2), c12, 2
            return chunk_score(ci), None, 1

        m_vec, c_vec, step = fold_pair(0)
        ci = step
        while ci < nchunks:
            v, c12, step = fold_pair(ci)
            if c12 is None:
                c12 = jnp.full((ROW_BLK, kb), ci, jnp.int32)
            better = v < m_vec
            m_vec = jnp.where(better, v, m_vec)
            if c_vec is None:
                c_vec = jnp.where(better, c12, jnp.int32(0))
            else:
                c_vec = jnp.where(better, c12, c_vec)
            ci += step
        m = jnp.min(m_vec, axis=1, keepdims=True)
        kcand = lax.broadcasted_iota(jnp.int32, (ROW_BLK, kb), 1)
        if c_vec is not None:
            kcand = c_vec * jnp.int32(kb) + kcand
        a = jnp.min(jnp.where(m_vec == m, kcand, jnp.int32(2 ** 30)),
                    axis=1, keepdims=True)
        idx_refs[j][...] = a


def _dist_multi(xf, ds, xn, dns):
    kdims = tuple(d.shape[1] for d in ds)
    nd = len(ds)
    return pl.pallas_call(
        functools.partial(_dist_multi_body, kdims=kdims),
        grid=(N_ROWS // ROW_BLK,),
        in_specs=(
            [pl.BlockSpec((ROW_BLK, EMB), lambda i: (i, 0))]
            + [pl.BlockSpec((EMB, k), lambda i: (0, 0)) for k in kdims]
            + [pl.BlockSpec((ROW_BLK, 1), lambda i: (i, 0))]
            + [pl.BlockSpec((1, k), lambda i: (0, 0)) for k in kdims]
        ),
        out_specs=[pl.BlockSpec((ROW_BLK, 1), lambda i: (i, 0))] * nd,
        out_shape=[jax.ShapeDtypeStruct((N_ROWS, 1), jnp.int32)] * nd,
    )(xf, *ds, xn, *dns)


def _sc_gather(table, idx):
    b_per_w = N_ROWS // SC_WORKERS
    mesh = plsc.VectorSubcoreMesh(core_axis_name="c", subcore_axis_name="s")

    @functools.partial(
        pl.kernel, mesh=mesh,
        out_type=jax.ShapeDtypeStruct((N_ROWS, EMB), jnp.float32),
        scratch_types=[
            pltpu.VMEM((SC_CHUNK,), jnp.int32),
            pltpu.VMEM((SC_CHUNK, EMB), jnp.float32),
            pltpu.SemaphoreType.DMA,
        ],
    )
    def k(table_hbm, idx_hbm, out_hbm, idx_v, rows_v, sem):
        wid = lax.axis_index("s") * 2 + lax.axis_index("c")
        base = wid * b_per_w
        for c in range(0, b_per_w, SC_CHUNK):
            pltpu.sync_copy(idx_hbm.at[pl.ds(base + c, SC_CHUNK)], idx_v)
            pltpu.async_copy(table_hbm.at[idx_v], rows_v, sem).wait()
            pltpu.sync_copy(rows_v, out_hbm.at[pl.ds(base + c, SC_CHUNK)])

    return k(table, idx)


def _combine_body(gam_ref, al_ref, x_ref, g0, g1, g2, g3, g4,
                  out_ref, loss_ref):
    i = pl.program_id(0)
    al = al_ref[0]
    x = x_ref[...]
    acc = jnp.zeros(x.shape, jnp.float32)
    lsum = jnp.float32(0.0)
    for j, g_ref in enumerate((g0, g1, g2, g3, g4)):
        q = al * g_ref[...].astype(jnp.bfloat16).astype(jnp.float32)
        acc = acc + gam_ref[j] * q
        dif = x - q
        lsum = lsum + gam_ref[j] * jnp.sum(dif * dif)
    out_ref[...] = acc

    @pl.when(i == 0)
    def _():
        loss_ref[...] = jnp.zeros((1, 1), jnp.float32)

    loss_ref[...] += jnp.reshape(lsum * ((1.0 + BETA) / (N_ROWS * EMB)),
                                 (1, 1))


def _combine(xf, gs, vq_gamma, vq_alpha):
    blk = 1024
    grid = (N_ROWS // blk,)
    row_spec = pl.BlockSpec((blk, EMB), lambda i: (i, 0))
    out, loss = pl.pallas_call(
        _combine_body,
        grid=grid,
        in_specs=[
            pl.BlockSpec(memory_space=pltpu.SMEM),
            pl.BlockSpec(memory_space=pltpu.SMEM),
            row_spec, row_spec, row_spec, row_spec, row_spec, row_spec,
        ],
        out_specs=[
            pl.BlockSpec((blk, EMB), lambda i: (i, 0)),
            pl.BlockSpec((1, 1), lambda i: (0, 0)),
        ],
        out_shape=[
            jax.ShapeDtypeStruct((N_ROWS, EMB), jnp.float32),
            jax.ShapeDtypeStruct((1, 1), jnp.float32),
        ],
    )(vq_gamma, vq_alpha, xf, *gs)
    return out, loss


def kernel(x, dict0, dict1, dict2, dict3, dict4, vq_alpha, vq_gamma):
    dicts = [dict0, dict1, dict2, dict3, dict4]
    xf = x.reshape(-1, EMB)
    xn = jnp.sum(xf ** 2, axis=1, keepdims=True)
    dns = [jnp.sum(d ** 2, axis=0, keepdims=True) for d in dicts]
    dbs = [d.astype(jnp.bfloat16) for d in dicts]
    idx4, idx3, idx2 = _dist_multi(
        xf, [dbs[4], dbs[3], dbs[2]], xn, [dns[4], dns[3], dns[2]])
    g4 = _sc_gather(dicts[4].T, idx4.reshape(N_ROWS))
    g3 = _sc_gather(dicts[3].T, idx3.reshape(N_ROWS))
    g2 = _sc_gather(dicts[2].T, idx2.reshape(N_ROWS))
    idx1, idx0 = _dist_multi(xf, [dbs[1], dbs[0]], xn, [dns[1], dns[0]])
    g1 = _sc_gather(dicts[1].T, idx1.reshape(N_ROWS))
    g0 = _sc_gather(dicts[0].T, idx0.reshape(N_ROWS))
    out, loss = _combine(xf, [g0, g1, g2, g3, g4], vq_gamma,
                         vq_alpha.reshape(1))
    return out.reshape(x.shape), loss[0, 0]


# speedup vs baseline: 1.0593x; 1.0024x over previous
import functools

import jax
import jax.numpy as jnp
from jax import lax
from jax.experimental import pallas as pl
from jax.experimental.pallas import tpu as pltpu
from jax.experimental.pallas import tpu_sc as plsc

EMB = 256
N_ROWS = 16384
BETA = 0.25

ROW_BLK = 512
K_BLK = 1024

SC_WORKERS = 32
SC_CHUNK = 256


def _dist_multi_body(x_ref, *refs, kdims):
    nd = len(kdims)
    d_refs = refs[:nd]
    xn_ref = refs[nd]
    dn_refs = refs[nd + 1:2 * nd + 1]
    idx_refs = refs[2 * nd + 1:]
    xm2 = (x_ref[...] * (-2.0)).astype(jnp.bfloat16)
    xn = xn_ref[...]
    for j in range(nd):
        kdim = kdims[j]
        kb = min(K_BLK, kdim)

        def chunk_score(ci):
            sim2 = lax.dot_general(
                xm2, d_refs[j][:, ci * kb:(ci + 1) * kb],
                (((1,), (0,)), ((), ())),
                preferred_element_type=jnp.float32)
            return (xn + dn_refs[j][0:1, ci * kb:(ci + 1) * kb]) + sim2

        nchunks = kdim // kb

        def fold_pair(ci):
            if ci + 1 < nchunks:
                v1 = chunk_score(ci)
                v2 = chunk_score(ci + 1)
                c12 = jnp.where(v2 < v1, jnp.int32(ci + 1), jnp.int32(ci))
                return jnp.minimum(v1, v2), c12, 2
            return chunk_score(ci), None, 1

        m_vec, c_vec, step = fold_pair(0)
        ci = step
        while ci < nchunks:
            v, c12, step = fold_pair(ci)
            if c12 is None:
                c12 = jnp.full((ROW_BLK, kb), ci, jnp.int32)
            better = v < m_vec
            m_vec = jnp.where(better, v, m_vec)
            if c_vec is None:
                c_vec = jnp.where(better, c12, jnp.int32(0))
            else:
                c_vec = jnp.where(better, c12, c_vec)
            ci += step
        m = jnp.min(m_vec, axis=1, keepdims=True)
        kcand = lax.broadcasted_iota(jnp.int32, (ROW_BLK, kb), 1)
        if c_vec is not None:
            kcand = c_vec * jnp.int32(kb) + kcand
        a = jnp.min(jnp.where(m_vec == m, kcand, jnp.int32(2 ** 30)),
                    axis=1, keepdims=True)
        idx_refs[j][...] = a


def _dist_multi(xf, ds, xn, dns, row0=0, nrows=N_ROWS):
    kdims = tuple(d.shape[1] for d in ds)
    nd = len(ds)
    boff = row0 // ROW_BLK
    return pl.pallas_call(
        functools.partial(_dist_multi_body, kdims=kdims),
        grid=(nrows // ROW_BLK,),
        in_specs=(
            [pl.BlockSpec((ROW_BLK, EMB), lambda i: (i + boff, 0))]
            + [pl.BlockSpec((EMB, k), lambda i: (0, 0)) for k in kdims]
            + [pl.BlockSpec((ROW_BLK, 1), lambda i: (i + boff, 0))]
            + [pl.BlockSpec((1, k), lambda i: (0, 0)) for k in kdims]
        ),
        out_specs=[pl.BlockSpec((ROW_BLK, 1), lambda i: (i, 0))] * nd,
        out_shape=[jax.ShapeDtypeStruct((nrows, 1), jnp.int32)] * nd,
    )(xf, *ds, xn, *dns)


def _sc_gather(table, idx):
    nrows = idx.shape[0]
    b_per_w = nrows // SC_WORKERS
    chunk = min(SC_CHUNK, b_per_w)
    mesh = plsc.VectorSubcoreMesh(core_axis_name="c", subcore_axis_name="s")

    @functools.partial(
        pl.kernel, mesh=mesh,
        out_type=jax.ShapeDtypeStruct((nrows, EMB), jnp.float32),
        scratch_types=[
            pltpu.VMEM((chunk,), jnp.int32),
            pltpu.VMEM((chunk, EMB), jnp.float32),
            pltpu.SemaphoreType.DMA,
        ],
    )
    def k(table_hbm, idx_hbm, out_hbm, idx_v, rows_v, sem):
        wid = lax.axis_index("s") * 2 + lax.axis_index("c")
        base = wid * b_per_w
        for c in range(0, b_per_w, chunk):
            pltpu.sync_copy(idx_hbm.at[pl.ds(base + c, chunk)], idx_v)
            pltpu.async_copy(table_hbm.at[idx_v], rows_v, sem).wait()
            pltpu.sync_copy(rows_v, out_hbm.at[pl.ds(base + c, chunk)])

    return k(table, idx)


def _combine_body(gam_ref, al_ref, x_ref, g0, g1, g2, g3, g4,
                  out_ref, loss_ref):
    i = pl.program_id(0)
    al = al_ref[0]
    x = x_ref[...]
    acc = jnp.zeros(x.shape, jnp.float32)
    lsum = jnp.float32(0.0)
    for j, g_ref in enumerate((g0, g1, g2, g3, g4)):
        q = al * g_ref[...].astype(jnp.bfloat16).astype(jnp.float32)
        acc = acc + gam_ref[j] * q
        dif = x - q
        lsum = lsum + gam_ref[j] * jnp.sum(dif * dif)
    out_ref[...] = acc

    @pl.when(i == 0)
    def _():
        loss_ref[...] = jnp.zeros((1, 1), jnp.float32)

    loss_ref[...] += jnp.reshape(lsum * ((1.0 + BETA) / (N_ROWS * EMB)),
                                 (1, 1))


def _combine(xf, gs, vq_gamma, vq_alpha, row0=0, nrows=N_ROWS):
    blk = 1024
    grid = (nrows // blk,)
    boff = row0 // blk

    def spec(g):
        off = boff if g.shape[0] == N_ROWS and nrows != N_ROWS else 0
        return pl.BlockSpec((blk, EMB), lambda i, o=off: (i + o, 0))

    out, loss = pl.pallas_call(
        _combine_body,
        grid=grid,
        in_specs=[
            pl.BlockSpec(memory_space=pltpu.SMEM),
            pl.BlockSpec(memory_space=pltpu.SMEM),
            pl.BlockSpec((blk, EMB), lambda i: (i + boff, 0)),
        ] + [spec(g) for g in gs],
        out_specs=[
            pl.BlockSpec((blk, EMB), lambda i: (i, 0)),
            pl.BlockSpec((1, 1), lambda i: (0, 0)),
        ],
        out_shape=[
            jax.ShapeDtypeStruct((nrows, EMB), jnp.float32),
            jax.ShapeDtypeStruct((1, 1), jnp.float32),
        ],
    )(vq_gamma, vq_alpha, xf, *gs)
    return out, loss


def kernel(x, dict0, dict1, dict2, dict3, dict4, vq_alpha, vq_gamma):
    dicts = [dict0, dict1, dict2, dict3, dict4]
    xf = x.reshape(-1, EMB)
    xn = jnp.sum(xf ** 2, axis=1, keepdims=True)
    dns = [jnp.sum(d ** 2, axis=0, keepdims=True) for d in dicts]
    dbs = [d.astype(jnp.bfloat16) for d in dicts]
    idx4, idx3, idx2 = _dist_multi(
        xf, [dbs[4], dbs[3], dbs[2]], xn, [dns[4], dns[3], dns[2]])
    g4 = _sc_gather(dicts[4].T, idx4.reshape(N_ROWS))
    g3 = _sc_gather(dicts[3].T, idx3.reshape(N_ROWS))
    g2 = _sc_gather(dicts[2].T, idx2.reshape(N_ROWS))
    half = N_ROWS // 2
    d1t, d0t = dicts[1].T, dicts[0].T
    al = vq_alpha.reshape(1)
    idx1a, idx0a = _dist_multi(xf, [dbs[1], dbs[0]], xn,
                               [dns[1], dns[0]], 0, half)
    g1a = _sc_gather(d1t, idx1a.reshape(half))
    g0a = _sc_gather(d0t, idx0a.reshape(half))
    idx1b, idx0b = _dist_multi(xf, [dbs[1], dbs[0]], xn,
                               [dns[1], dns[0]], half, half)
    g1b = _sc_gather(d1t, idx1b.reshape(half))
    g0b = _sc_gather(d0t, idx0b.reshape(half))
    out_a, loss_a = _combine(xf, [g0a, g1a, g2, g3, g4], vq_gamma, al,
                             0, half)
    out_b, loss_b = _combine(xf, [g0b, g1b, g2, g3, g4], vq_gamma, al,
                             half, half)
    out = jnp.concatenate([out_a, out_b], axis=0)
    return out.reshape(x.shape), loss_a[0, 0] + loss_b[0, 0]


# speedup vs baseline: 1.0842x; 1.0235x over previous
import functools

import jax
import jax.numpy as jnp
from jax import lax
from jax.experimental import pallas as pl
from jax.experimental.pallas import tpu as pltpu
from jax.experimental.pallas import tpu_sc as plsc

EMB = 256
N_ROWS = 16384
BETA = 0.25

ROW_BLK = 512
K_BLK = 1024

SC_WORKERS = 32
SC_CHUNK = 256


def _dist_multi_body(x_ref, *refs, kdims):
    nd = len(kdims)
    d_refs = refs[:nd]
    xn_ref = refs[nd]
    dn_refs = refs[nd + 1:2 * nd + 1]
    idx_refs = refs[2 * nd + 1:]
    xm2 = (x_ref[...] * (-2.0)).astype(jnp.bfloat16)
    xn = xn_ref[...]
    for j in range(nd):
        kdim = kdims[j]
        kb = min(K_BLK, kdim)

        def chunk_score(ci):
            sim2 = lax.dot_general(
                xm2, d_refs[j][:, ci * kb:(ci + 1) * kb],
                (((1,), (0,)), ((), ())),
                preferred_element_type=jnp.float32)
            return (xn + dn_refs[j][0:1, ci * kb:(ci + 1) * kb]) + sim2

        nchunks = kdim // kb

        def fold_pair(ci):
            if ci + 1 < nchunks:
                v1 = chunk_score(ci)
                v2 = chunk_score(ci + 1)
                c12 = jnp.where(v2 < v1, jnp.int32(ci + 1), jnp.int32(ci))
                return jnp.minimum(v1, v2), c12, 2
            return chunk_score(ci), None, 1

        m_vec, c_vec, step = fold_pair(0)
        ci = step
        while ci < nchunks:
            v, c12, step = fold_pair(ci)
            if c12 is None:
                c12 = jnp.full((ROW_BLK, kb), ci, jnp.int32)
            better = v < m_vec
            m_vec = jnp.where(better, v, m_vec)
            if c_vec is None:
                c_vec = jnp.where(better, c12, jnp.int32(0))
            else:
                c_vec = jnp.where(better, c12, c_vec)
            ci += step
        m = jnp.min(m_vec, axis=1, keepdims=True)
        kcand = lax.broadcasted_iota(jnp.int32, (ROW_BLK, kb), 1)
        if c_vec is not None:
            kcand = c_vec * jnp.int32(kb) + kcand
        a = jnp.min(jnp.where(m_vec == m, kcand, jnp.int32(2 ** 30)),
                    axis=1, keepdims=True)
        idx_refs[j][...] = a


def _dist_multi(xf, ds, xn, dns, row0=0, nrows=N_ROWS):
    kdims = tuple(d.shape[1] for d in ds)
    nd = len(ds)
    boff = row0 // ROW_BLK
    return pl.pallas_call(
        functools.partial(_dist_multi_body, kdims=kdims),
        grid=(nrows // ROW_BLK,),
        in_specs=(
            [pl.BlockSpec((ROW_BLK, EMB), lambda i: (i + boff, 0))]
            + [pl.BlockSpec((EMB, k), lambda i: (0, 0)) for k in kdims]
            + [pl.BlockSpec((ROW_BLK, 1), lambda i: (i + boff, 0))]
            + [pl.BlockSpec((1, k), lambda i: (0, 0)) for k in kdims]
        ),
        out_specs=[pl.BlockSpec((ROW_BLK, 1), lambda i: (i, 0))] * nd,
        out_shape=[jax.ShapeDtypeStruct((nrows, 1), jnp.int32)] * nd,
    )(xf, *ds, xn, *dns)


def _sc_gather(table, idx):
    nrows = idx.shape[0]
    b_per_w = nrows // SC_WORKERS
    chunk = min(SC_CHUNK, b_per_w)
    mesh = plsc.VectorSubcoreMesh(core_axis_name="c", subcore_axis_name="s")

    @functools.partial(
        pl.kernel, mesh=mesh,
        out_type=jax.ShapeDtypeStruct((nrows, EMB), jnp.float32),
        scratch_types=[
            pltpu.VMEM((chunk,), jnp.int32),
            pltpu.VMEM((chunk, EMB), jnp.float32),
            pltpu.SemaphoreType.DMA,
        ],
    )
    def k(table_hbm, idx_hbm, out_hbm, idx_v, rows_v, sem):
        wid = lax.axis_index("s") * 2 + lax.axis_index("c")
        base = wid * b_per_w
        for c in range(0, b_per_w, chunk):
            pltpu.sync_copy(idx_hbm.at[pl.ds(base + c, chunk)], idx_v)
            pltpu.async_copy(table_hbm.at[idx_v], rows_v, sem).wait()
            pltpu.sync_copy(rows_v, out_hbm.at[pl.ds(base + c, chunk)])

    return k(table, idx)


def _combine_body(gam_ref, al_ref, x_ref, g0, g1, g2, g3, g4, *rest):
    out_ref, loss_ref = rest[-2], rest[-1]
    i = pl.program_id(0)
    al = al_ref[0]
    x = x_ref[...]
    acc = jnp.zeros(x.shape, jnp.float32)
    lsum = jnp.float32(0.0)
    for j, g_ref in enumerate((g0, g1, g2, g3, g4)):
        q = al * g_ref[...].astype(jnp.bfloat16).astype(jnp.float32)
        acc = acc + gam_ref[j] * q
        dif = x - q
        lsum = lsum + gam_ref[j] * jnp.sum(dif * dif)
    out_ref[...] = acc

    @pl.when(i == 0)
    def _():
        loss_ref[...] = jnp.zeros((1, 1), jnp.float32)

    loss_ref[...] += jnp.reshape(lsum * ((1.0 + BETA) / (N_ROWS * EMB)),
                                 (1, 1))


def _combine(xf, gs, vq_gamma, vq_alpha, row0=0, nrows=N_ROWS, prev=None):
    blk = 1024
    grid = (nrows // blk,)
    boff = row0 // blk

    def spec(g):
        off = boff if g.shape[0] == N_ROWS and nrows != N_ROWS else 0
        return pl.BlockSpec((blk, EMB), lambda i, o=off: (i + o, 0))

    in_specs = [
        pl.BlockSpec(memory_space=pltpu.SMEM),
        pl.BlockSpec(memory_space=pltpu.SMEM),
        pl.BlockSpec((blk, EMB), lambda i: (i + boff, 0)),
    ] + [spec(g) for g in gs]
    args = [vq_gamma, vq_alpha, xf] + list(gs)
    aliases = {}
    if prev is not None:
        in_specs.append(pl.BlockSpec(memory_space=pl.ANY))
        aliases = {len(args): 0}
        args.append(prev)
    out, loss = pl.pallas_call(
        _combine_body,
        grid=grid,
        in_specs=in_specs,
        out_specs=[
            pl.BlockSpec((blk, EMB), lambda i: (i + boff, 0)),
            pl.BlockSpec((1, 1), lambda i: (0, 0)),
        ],
        out_shape=[
            jax.ShapeDtypeStruct((N_ROWS, EMB), jnp.float32),
            jax.ShapeDtypeStruct((1, 1), jnp.float32),
        ],
        input_output_aliases=aliases,
    )(*args)
    return out, loss


def kernel(x, dict0, dict1, dict2, dict3, dict4, vq_alpha, vq_gamma):
    dicts = [dict0, dict1, dict2, dict3, dict4]
    xf = x.reshape(-1, EMB)
    xn = jnp.sum(xf ** 2, axis=1, keepdims=True)
    dns = [jnp.sum(d ** 2, axis=0, keepdims=True) for d in dicts]
    dbs = [d.astype(jnp.bfloat16) for d in dicts]
    idx4, idx3, idx2 = _dist_multi(
        xf, [dbs[4], dbs[3], dbs[2]], xn, [dns[4], dns[3], dns[2]])
    g4 = _sc_gather(dicts[4].T, idx4.reshape(N_ROWS))
    g3 = _sc_gather(dicts[3].T, idx3.reshape(N_ROWS))
    g2 = _sc_gather(dicts[2].T, idx2.reshape(N_ROWS))
    half = N_ROWS // 2
    d1t, d0t = dicts[1].T, dicts[0].T
    al = vq_alpha.reshape(1)
    idx1a, idx0a = _dist_multi(xf, [dbs[1], dbs[0]], xn,
                               [dns[1], dns[0]], 0, half)
    g1a = _sc_gather(d1t, idx1a.reshape(half))
    g0a = _sc_gather(d0t, idx0a.reshape(half))
    idx1b, idx0b = _dist_multi(xf, [dbs[1], dbs[0]], xn,
                               [dns[1], dns[0]], half, half)
    g1b = _sc_gather(d1t, idx1b.reshape(half))
    g0b = _sc_gather(d0t, idx0b.reshape(half))
    out_a, loss_a = _combine(xf, [g0a, g1a, g2, g3, g4], vq_gamma, al,
                             0, half)
    out, loss_b = _combine(xf, [g0b, g1b, g2, g3, g4], vq_gamma, al,
                           half, half, prev=out_a)
    return out.reshape(x.shape), loss_a[0, 0] + loss_b[0, 0]


# speedup vs baseline: 1.1211x; 1.0340x over previous
import functools

import jax
import jax.numpy as jnp
from jax import lax
from jax.experimental import pallas as pl
from jax.experimental.pallas import tpu as pltpu
from jax.experimental.pallas import tpu_sc as plsc

EMB = 256
N_ROWS = 16384
BETA = 0.25

ROW_BLK = 512
K_BLK = 512

SC_WORKERS = 32
SC_CHUNK = 256


def _dist_multi_body(x_ref, *refs, kdims):
    nd = len(kdims)
    d_refs = refs[:nd]
    xn_ref = refs[nd]
    dn_refs = refs[nd + 1:2 * nd + 1]
    idx_refs = refs[2 * nd + 1:]
    xm2 = (x_ref[...] * (-2.0)).astype(jnp.bfloat16)
    xn = xn_ref[...]
    for j in range(nd):
        kdim = kdims[j]
        kb = min(K_BLK, kdim)

        def chunk_score(ci):
            sim2 = lax.dot_general(
                xm2, d_refs[j][:, ci * kb:(ci + 1) * kb],
                (((1,), (0,)), ((), ())),
                preferred_element_type=jnp.float32)
            return (xn + dn_refs[j][0:1, ci * kb:(ci + 1) * kb]) + sim2

        nchunks = kdim // kb

        def fold_pair(ci):
            if ci + 1 < nchunks:
                v1 = chunk_score(ci)
                v2 = chunk_score(ci + 1)
                c12 = jnp.where(v2 < v1, jnp.int32(ci + 1), jnp.int32(ci))
                return jnp.minimum(v1, v2), c12, 2
            return chunk_score(ci), None, 1

        m_vec, c_vec, step = fold_pair(0)
        ci = step
        while ci < nchunks:
            v, c12, step = fold_pair(ci)
            if c12 is None:
                c12 = jnp.full((ROW_BLK, kb), ci, jnp.int32)
            better = v < m_vec
            m_vec = jnp.where(better, v, m_vec)
            if c_vec is None:
                c_vec = jnp.where(better, c12, jnp.int32(0))
            else:
                c_vec = jnp.where(better, c12, c_vec)
            ci += step
        m = jnp.min(m_vec, axis=1, keepdims=True)
        kcand = lax.broadcasted_iota(jnp.int32, (ROW_BLK, kb), 1)
        if c_vec is not None:
            kcand = c_vec * jnp.int32(kb) + kcand
        a = jnp.min(jnp.where(m_vec == m, kcand, jnp.int32(2 ** 30)),
                    axis=1, keepdims=True)
        idx_refs[j][...] = a


def _dist_multi(xf, ds, xn, dns, row0=0, nrows=N_ROWS):
    kdims = tuple(d.shape[1] for d in ds)
    nd = len(ds)
    boff = row0 // ROW_BLK
    return pl.pallas_call(
        functools.partial(_dist_multi_body, kdims=kdims),
        grid=(nrows // ROW_BLK,),
        in_specs=(
            [pl.BlockSpec((ROW_BLK, EMB), lambda i: (i + boff, 0))]
            + [pl.BlockSpec((EMB, k), lambda i: (0, 0)) for k in kdims]
            + [pl.BlockSpec((ROW_BLK, 1), lambda i: (i + boff, 0))]
            + [pl.BlockSpec((1, k), lambda i: (0, 0)) for k in kdims]
        ),
        out_specs=[pl.BlockSpec((ROW_BLK, 1), lambda i: (i, 0))] * nd,
        out_shape=[jax.ShapeDtypeStruct((nrows, 1), jnp.int32)] * nd,
    )(xf, *ds, xn, *dns)


def _sc_gather(table, idx):
    nrows = idx.shape[0]
    b_per_w = nrows // SC_WORKERS
    chunk = min(SC_CHUNK, b_per_w)
    mesh = plsc.VectorSubcoreMesh(core_axis_name="c", subcore_axis_name="s")

    @functools.partial(
        pl.kernel, mesh=mesh,
        out_type=jax.ShapeDtypeStruct((nrows, EMB), jnp.float32),
        scratch_types=[
            pltpu.VMEM((chunk,), jnp.int32),
            pltpu.VMEM((chunk, EMB), jnp.float32),
            pltpu.SemaphoreType.DMA,
        ],
    )
    def k(table_hbm, idx_hbm, out_hbm, idx_v, rows_v, sem):
        wid = lax.axis_index("s") * 2 + lax.axis_index("c")
        base = wid * b_per_w
        for c in range(0, b_per_w, chunk):
            pltpu.sync_copy(idx_hbm.at[pl.ds(base + c, chunk)], idx_v)
            pltpu.async_copy(table_hbm.at[idx_v], rows_v, sem).wait()
            pltpu.sync_copy(rows_v, out_hbm.at[pl.ds(base + c, chunk)])

    return k(table, idx)


def _combine_body(gam_ref, al_ref, x_ref, g0, g1, g2, g3, g4, *rest):
    out_ref, loss_ref = rest[-2], rest[-1]
    i = pl.program_id(0)
    al = al_ref[0]
    x = x_ref[...]
    acc = jnp.zeros(x.shape, jnp.float32)
    lsum = jnp.float32(0.0)
    for j, g_ref in enumerate((g0, g1, g2, g3, g4)):
        q = al * g_ref[...].astype(jnp.bfloat16).astype(jnp.float32)
        acc = acc + gam_ref[j] * q
        dif = x - q
        lsum = lsum + gam_ref[j] * jnp.sum(dif * dif)
    out_ref[...] = acc

    @pl.when(i == 0)
    def _():
        loss_ref[...] = jnp.zeros((1, 1), jnp.float32)

    loss_ref[...] += jnp.reshape(lsum * ((1.0 + BETA) / (N_ROWS * EMB)),
                                 (1, 1))


def _combine(xf, gs, vq_gamma, vq_alpha, row0=0, nrows=N_ROWS, prev=None):
    blk = 1024
    grid = (nrows // blk,)
    boff = row0 // blk

    def spec(g):
        off = boff if g.shape[0] == N_ROWS and nrows != N_ROWS else 0
        return pl.BlockSpec((blk, EMB), lambda i, o=off: (i + o, 0))

    in_specs = [
        pl.BlockSpec(memory_space=pltpu.SMEM),
        pl.BlockSpec(memory_space=pltpu.SMEM),
        pl.BlockSpec((blk, EMB), lambda i: (i + boff, 0)),
    ] + [spec(g) for g in gs]
    args = [vq_gamma, vq_alpha, xf] + list(gs)
    aliases = {}
    if prev is not None:
        in_specs.append(pl.BlockSpec(memory_space=pl.ANY))
        aliases = {len(args): 0}
        args.append(prev)
    out, loss = pl.pallas_call(
        _combine_body,
        grid=grid,
        in_specs=in_specs,
        out_specs=[
            pl.BlockSpec((blk, EMB), lambda i: (i + boff, 0)),
            pl.BlockSpec((1, 1), lambda i: (0, 0)),
        ],
        out_shape=[
            jax.ShapeDtypeStruct((N_ROWS, EMB), jnp.float32),
            jax.ShapeDtypeStruct((1, 1), jnp.float32),
        ],
        input_output_aliases=aliases,
    )(*args)
    return out, loss


def kernel(x, dict0, dict1, dict2, dict3, dict4, vq_alpha, vq_gamma):
    dicts = [dict0, dict1, dict2, dict3, dict4]
    xf = x.reshape(-1, EMB)
    xn = jnp.sum(xf ** 2, axis=1, keepdims=True)
    dns = [jnp.sum(d ** 2, axis=0, keepdims=True) for d in dicts]
    dbs = [d.astype(jnp.bfloat16) for d in dicts]
    idx4, idx3, idx2 = _dist_multi(
        xf, [dbs[4], dbs[3], dbs[2]], xn, [dns[4], dns[3], dns[2]])
    g4 = _sc_gather(dicts[4].T, idx4.reshape(N_ROWS))
    g3 = _sc_gather(dicts[3].T, idx3.reshape(N_ROWS))
    g2 = _sc_gather(dicts[2].T, idx2.reshape(N_ROWS))
    half = N_ROWS // 2
    d1t, d0t = dicts[1].T, dicts[0].T
    al = vq_alpha.reshape(1)
    idx1a, idx0a = _dist_multi(xf, [dbs[1], dbs[0]], xn,
                               [dns[1], dns[0]], 0, half)
    g1a = _sc_gather(d1t, idx1a.reshape(half))
    g0a = _sc_gather(d0t, idx0a.reshape(half))
    idx1b, idx0b = _dist_multi(xf, [dbs[1], dbs[0]], xn,
                               [dns[1], dns[0]], half, half)
    g1b = _sc_gather(d1t, idx1b.reshape(half))
    g0b = _sc_gather(d0t, idx0b.reshape(half))
    out_a, loss_a = _combine(xf, [g0a, g1a, g2, g3, g4], vq_gamma, al,
                             0, half)
    out, loss_b = _combine(xf, [g0b, g1b, g2, g3, g4], vq_gamma, al,
                           half, half, prev=out_a)
    return out.reshape(x.shape), loss_a[0, 0] + loss_b[0, 0]


# speedup vs baseline: 1.2349x; 1.1015x over previous
import functools

import jax
import jax.numpy as jnp
from jax import lax
from jax.experimental import pallas as pl
from jax.experimental.pallas import tpu as pltpu
from jax.experimental.pallas import tpu_sc as plsc

EMB = 256
N_ROWS = 16384
BETA = 0.25

ROW_BLK = 512
K_BLK = 256

SC_WORKERS = 32
SC_CHUNK = 256


def _dist_multi_body(x_ref, *refs, kdims):
    nd = len(kdims)
    d_refs = refs[:nd]
    xn_ref = refs[nd]
    dn_refs = refs[nd + 1:2 * nd + 1]
    idx_refs = refs[2 * nd + 1:]
    xm2 = (x_ref[...] * (-2.0)).astype(jnp.bfloat16)
    xn = xn_ref[...]
    for j in range(nd):
        kdim = kdims[j]
        kb = min(K_BLK, kdim)

        def chunk_score(ci):
            sim2 = lax.dot_general(
                xm2, d_refs[j][:, ci * kb:(ci + 1) * kb],
                (((1,), (0,)), ((), ())),
                preferred_element_type=jnp.float32)
            return (xn + dn_refs[j][0:1, ci * kb:(ci + 1) * kb]) + sim2

        nchunks = kdim // kb

        def fold_pair(ci):
            if ci + 1 < nchunks:
                v1 = chunk_score(ci)
                v2 = chunk_score(ci + 1)
                c12 = jnp.where(v2 < v1, jnp.int32(ci + 1), jnp.int32(ci))
                return jnp.minimum(v1, v2), c12, 2
            return chunk_score(ci), None, 1

        m_vec, c_vec, step = fold_pair(0)
        ci = step
        while ci < nchunks:
            v, c12, step = fold_pair(ci)
            if c12 is None:
                c12 = jnp.full((ROW_BLK, kb), ci, jnp.int32)
            better = v < m_vec
            m_vec = jnp.where(better, v, m_vec)
            if c_vec is None:
                c_vec = jnp.where(better, c12, jnp.int32(0))
            else:
                c_vec = jnp.where(better, c12, c_vec)
            ci += step
        m = jnp.min(m_vec, axis=1, keepdims=True)
        kcand = lax.broadcasted_iota(jnp.int32, (ROW_BLK, kb), 1)
        if c_vec is not None:
            kcand = c_vec * jnp.int32(kb) + kcand
        a = jnp.min(jnp.where(m_vec == m, kcand, jnp.int32(2 ** 30)),
                    axis=1, keepdims=True)
        idx_refs[j][...] = a


def _dist_multi(xf, ds, xn, dns, row0=0, nrows=N_ROWS):
    kdims = tuple(d.shape[1] for d in ds)
    nd = len(ds)
    boff = row0 // ROW_BLK
    return pl.pallas_call(
        functools.partial(_dist_multi_body, kdims=kdims),
        grid=(nrows // ROW_BLK,),
        in_specs=(
            [pl.BlockSpec((ROW_BLK, EMB), lambda i: (i + boff, 0))]
            + [pl.BlockSpec((EMB, k), lambda i: (0, 0)) for k in kdims]
            + [pl.BlockSpec((ROW_BLK, 1), lambda i: (i + boff, 0))]
            + [pl.BlockSpec((1, k), lambda i: (0, 0)) for k in kdims]
        ),
        out_specs=[pl.BlockSpec((ROW_BLK, 1), lambda i: (i, 0))] * nd,
        out_shape=[jax.ShapeDtypeStruct((nrows, 1), jnp.int32)] * nd,
    )(xf, *ds, xn, *dns)


def _sc_gather(table, idx):
    nrows = idx.shape[0]
    b_per_w = nrows // SC_WORKERS
    chunk = min(SC_CHUNK, b_per_w)
    mesh = plsc.VectorSubcoreMesh(core_axis_name="c", subcore_axis_name="s")

    @functools.partial(
        pl.kernel, mesh=mesh,
        out_type=jax.ShapeDtypeStruct((nrows, EMB), jnp.float32),
        scratch_types=[
            pltpu.VMEM((chunk,), jnp.int32),
            pltpu.VMEM((chunk, EMB), jnp.float32),
            pltpu.SemaphoreType.DMA,
        ],
    )
    def k(table_hbm, idx_hbm, out_hbm, idx_v, rows_v, sem):
        wid = lax.axis_index("s") * 2 + lax.axis_index("c")
        base = wid * b_per_w
        for c in range(0, b_per_w, chunk):
            pltpu.sync_copy(idx_hbm.at[pl.ds(base + c, chunk)], idx_v)
            pltpu.async_copy(table_hbm.at[idx_v], rows_v, sem).wait()
            pltpu.sync_copy(rows_v, out_hbm.at[pl.ds(base + c, chunk)])

    return k(table, idx)


def _combine_body(gam_ref, al_ref, x_ref, g0, g1, g2, g3, g4, *rest):
    out_ref, loss_ref = rest[-2], rest[-1]
    i = pl.program_id(0)
    al = al_ref[0]
    x = x_ref[...]
    acc = jnp.zeros(x.shape, jnp.float32)
    lsum = jnp.float32(0.0)
    for j, g_ref in enumerate((g0, g1, g2, g3, g4)):
        q = al * g_ref[...].astype(jnp.bfloat16).astype(jnp.float32)
        acc = acc + gam_ref[j] * q
        dif = x - q
        lsum = lsum + gam_ref[j] * jnp.sum(dif * dif)
    out_ref[...] = acc

    @pl.when(i == 0)
    def _():
        loss_ref[...] = jnp.zeros((1, 1), jnp.float32)

    loss_ref[...] += jnp.reshape(lsum * ((1.0 + BETA) / (N_ROWS * EMB)),
                                 (1, 1))


def _combine(xf, gs, vq_gamma, vq_alpha, row0=0, nrows=N_ROWS, prev=None):
    blk = 1024
    grid = (nrows // blk,)
    boff = row0 // blk

    def spec(g):
        off = boff if g.shape[0] == N_ROWS and nrows != N_ROWS else 0
        return pl.BlockSpec((blk, EMB), lambda i, o=off: (i + o, 0))

    in_specs = [
        pl.BlockSpec(memory_space=pltpu.SMEM),
        pl.BlockSpec(memory_space=pltpu.SMEM),
        pl.BlockSpec((blk, EMB), lambda i: (i + boff, 0)),
    ] + [spec(g) for g in gs]
    args = [vq_gamma, vq_alpha, xf] + list(gs)
    aliases = {}
    if prev is not None:
        in_specs.append(pl.BlockSpec(memory_space=pl.ANY))
        aliases = {len(args): 0}
        args.append(prev)
    out, loss = pl.pallas_call(
        _combine_body,
        grid=grid,
        in_specs=in_specs,
        out_specs=[
            pl.BlockSpec((blk, EMB), lambda i: (i + boff, 0)),
            pl.BlockSpec((1, 1), lambda i: (0, 0)),
        ],
        out_shape=[
            jax.ShapeDtypeStruct((N_ROWS, EMB), jnp.float32),
            jax.ShapeDtypeStruct((1, 1), jnp.float32),
        ],
        input_output_aliases=aliases,
    )(*args)
    return out, loss


def kernel(x, dict0, dict1, dict2, dict3, dict4, vq_alpha, vq_gamma):
    dicts = [dict0, dict1, dict2, dict3, dict4]
    xf = x.reshape(-1, EMB)
    xn = jnp.sum(xf ** 2, axis=1, keepdims=True)
    dns = [jnp.sum(d ** 2, axis=0, keepdims=True) for d in dicts]
    dbs = [d.astype(jnp.bfloat16) for d in dicts]
    idx4, idx3, idx2 = _dist_multi(
        xf, [dbs[4], dbs[3], dbs[2]], xn, [dns[4], dns[3], dns[2]])
    g4 = _sc_gather(dicts[4].T, idx4.reshape(N_ROWS))
    g3 = _sc_gather(dicts[3].T, idx3.reshape(N_ROWS))
    g2 = _sc_gather(dicts[2].T, idx2.reshape(N_ROWS))
    half = N_ROWS // 2
    d1t, d0t = dicts[1].T, dicts[0].T
    al = vq_alpha.reshape(1)
    idx1a, idx0a = _dist_multi(xf, [dbs[1], dbs[0]], xn,
                               [dns[1], dns[0]], 0, half)
    g1a = _sc_gather(d1t, idx1a.reshape(half))
    g0a = _sc_gather(d0t, idx0a.reshape(half))
    idx1b, idx0b = _dist_multi(xf, [dbs[1], dbs[0]], xn,
                               [dns[1], dns[0]], half, half)
    g1b = _sc_gather(d1t, idx1b.reshape(half))
    g0b = _sc_gather(d0t, idx0b.reshape(half))
    out_a, loss_a = _combine(xf, [g0a, g1a, g2, g3, g4], vq_gamma, al,
                             0, half)
    out, loss_b = _combine(xf, [g0b, g1b, g2, g3, g4], vq_gamma, al,
                           half, half, prev=out_a)
    return out.reshape(x.shape), loss_a[0, 0] + loss_b[0, 0]
